# baseline (speedup 1.0000x reference)
import jax
import jax.numpy as jnp
from jax import lax
from jax.experimental import pallas as pl
from jax.experimental.pallas import tpu as pltpu
from jax.experimental.pallas import tpu_sc as plsc

T = 2048
D = 2048
F = 1024
E = 8
K = 2
C = 2 * T * K // E

NC, NS, L = 2, 16, 16
NW = NC * NS

BT = 256
BC = 256
NCB = C // BC

_STAGE = 128
_DUMP = E * C


def _routing_body(x_ref, rw_ref, topi_ref, topw_ref):
    lt = lax.dot_general(rw_ref[...], x_ref[...],
                         (((0,), (1,)), ((), ())),
                         preferred_element_type=jnp.float32)
    row = lax.broadcasted_iota(jnp.int32, (E, BT), 0)
    m1 = jnp.max(lt, axis=0, keepdims=True)
    i1 = jnp.min(jnp.where(lt == m1, row, E), axis=0, keepdims=True)
    l2 = jnp.where(row == i1, -3e38, lt)
    m2 = jnp.max(l2, axis=0, keepdims=True)
    i2 = jnp.min(jnp.where(l2 == m2, row, E), axis=0, keepdims=True)
    r = jnp.exp(m2 - m1)
    w1 = 1.0 / (1.0 + r)
    w2 = r / (1.0 + r)
    topi_ref[...] = jnp.where(row == 0, i1, jnp.where(row == 1, i2, 0))
    topw_ref[...] = jnp.where(row == 0, w1, jnp.where(row == 1, w2, 0.0))


def _routing(x, router_w):
    return pl.pallas_call(
        _routing_body,
        grid=(T // BT,),
        in_specs=[
            pl.BlockSpec((BT, D), lambda i: (i, 0)),
            pl.BlockSpec((D, E), lambda i: (0, 0)),
        ],
        out_specs=[
            pl.BlockSpec((E, BT), lambda i: (0, i)),
            pl.BlockSpec((E, BT), lambda i: (0, i)),
        ],
        out_shape=[
            jax.ShapeDtypeStruct((E, T), jnp.int32),
            jax.ShapeDtypeStruct((E, T), jnp.float32),
        ],
    )(x, router_w)


def _dg(v, idx):
    return lax.gather(
        v, idx[:, None],
        lax.GatherDimensionNumbers(
            offset_dims=(), collapsed_slice_dims=(0,), start_index_map=(0,)),
        (1,), mode=lax.GatherScatterMode.PROMISE_IN_BOUNDS)


def _dispatch_body(topi_hbm, topw_hbm, idx_hbm, wslot_hbm, spair_hbm,
                   cnt_hbm, topi_v, topw_v, spair_v, zi_v, zf_v,
                   slot_st, tok_st, w_st, outv_v, sem):
    wid = lax.axis_index("s") * NC + lax.axis_index("c")

    @pl.when(wid == 0)
    def _():
        pltpu.sync_copy(topi_hbm.at[pl.ds(0, K)], topi_v)
        pltpu.sync_copy(topw_hbm.at[pl.ds(0, K)], topw_v)

        lanes = jnp.arange(L, dtype=jnp.int32)
        half = lanes >> 1
        parity_odd = (lanes & 1) == 1
        shift_idx = [jnp.maximum(lanes - d, 0) for d in (1, 2, 4, 8)]
        shift_ok = [lanes >= d for d in (1, 2, 4, 8)]
        bfly_idx = [lanes ^ d for d in (1, 2, 4, 8)]
        last_idx = jnp.full((L,), L - 1, jnp.int32)

        def zinit(i, c):
            zi_v[pl.ds(i * L, L)] = jnp.zeros((L,), jnp.int32)
            zf_v[pl.ds(i * L, L)] = jnp.zeros((L,), jnp.float32)
            return c

        lax.fori_loop(0, C // L, zinit, 0)

        def zout(i, c):
            pltpu.sync_copy(zi_v, idx_hbm.at[pl.ds(i * C, C)])
            pltpu.sync_copy(zf_v, wslot_hbm.at[pl.ds(i * C, C)])
            return c

        lax.fori_loop(0, E, zout, 0)

        def outer(m, cnt):
            ea = topi_v[0, pl.ds(m * L, L)]
            eb = topi_v[1, pl.ds(m * L, L)]
            wa = topw_v[0, pl.ds(m * L, L)]
            wb = topw_v[1, pl.ds(m * L, L)]
            for h in range(2):
                p = half + (h * (L // 2))
                e_vec = jnp.where(parity_odd, _dg(eb, p), _dg(ea, p))
                w_vec = jnp.where(parity_odd, _dg(wb, p), _dg(wa, p))
                t_vec = m * L + h * (L // 2) + half
                base = _dg(cnt, e_vec)
                pos = jnp.zeros((L,), jnp.int32)
                for x in range(E):
                    mx = e_vec == x
                    s = jnp.where(mx, 1, 0)
                    for d in range(4):
                        s = s + jnp.where(shift_ok[d], _dg(s, shift_idx[d]), 0)
                    pos = jnp.where(mx, s - 1, pos)
                    cnt = cnt + jnp.where(lanes == x, _dg(s, last_idx), 0)
                pos = base + pos
                valid = pos < C
                slot = e_vec * C + pos
                q = (m & 3) * (2 * L) + h * L
                slot_st[pl.ds(q, L)] = jnp.where(valid, slot, _DUMP + lanes)
                tok_st[pl.ds(q, L)] = t_vec
                w_st[pl.ds(q, L)] = w_vec
                spair_v[pl.ds(m * (2 * L) + h * L, L)] = (
                    jnp.where(valid, slot, -1))

            @pl.when((m & 3) == 3)
            def _():
                pltpu.async_copy(tok_st, idx_hbm.at[slot_st], sem).wait()
                pltpu.async_copy(w_st, wslot_hbm.at[slot_st], sem).wait()

            return cnt

        cnt = lax.fori_loop(0, T // L, outer,
                            jnp.zeros((L,), jnp.int32))

        cv = jnp.minimum(cnt, C)
        outv_v[...] = cv
        pltpu.sync_copy(outv_v, cnt_hbm)

        def fix(j, c):
            sp = spair_v[pl.ds(j * L, L)]
            spair_v[pl.ds(j * L, L)] = jnp.where(sp < 0, E * C, sp)
            return c

        lax.fori_loop(0, (T * K) // L, fix, 0)
        pltpu.sync_copy(spair_v, spair_hbm)


def _dispatch(topi, topw):
    mesh = plsc.VectorSubcoreMesh(core_axis_name="c", subcore_axis_name="s")
    f = pl.kernel(
        _dispatch_body,
        mesh=mesh,
        out_type=[
            jax.ShapeDtypeStruct((E * C + L,), jnp.int32),
            jax.ShapeDtypeStruct((E * C + L,), jnp.float32),
            jax.ShapeDtypeStruct((T * K,), jnp.int32),
            jax.ShapeDtypeStruct((L,), jnp.int32),
        ],
        scratch_types=[
            pltpu.VMEM((K, T), jnp.int32),
            pltpu.VMEM((K, T), jnp.float32),
            pltpu.VMEM((T * K,), jnp.int32),
            pltpu.VMEM((C,), jnp.int32),
            pltpu.VMEM((C,), jnp.float32),
            pltpu.VMEM((_STAGE,), jnp.int32),
            pltpu.VMEM((_STAGE,), jnp.int32),
            pltpu.VMEM((_STAGE,), jnp.float32),
            pltpu.VMEM((L,), jnp.int32),
            pltpu.SemaphoreType.DMA,
        ],
    )
    return f(topi, topw)


_GROWS = (E * C) // NW
_GCHUNK = 16
_GNCH = _GROWS // _GCHUNK


def _gather_body(x_hbm, idx_hbm, cnt_hbm, g_hbm, idx_v, cv_v,
                 rows0, rows1, sem0, sem1):
    wid = lax.axis_index("s") * NC + lax.axis_index("c")
    pltpu.sync_copy(cnt_hbm, cv_v)
    myexp = wid >> 2
    jrot = ((wid & 3) + myexp) & 3
    base = myexp * C + jrot * _GROWS
    mybase = jrot * _GROWS
    cv = cv_v[pl.ds(0, L)]
    my_cnt = cv[0]
    for i in range(1, E):
        my_cnt = jnp.where(myexp == i, cv[i], my_cnt)
    active = my_cnt > mybase

    @pl.when(active)
    def _():
        pltpu.sync_copy(idx_hbm.at[pl.ds(base, _GROWS)], idx_v)
        bufs = (rows0, rows1)
        sems = (sem0, sem1)
        pltpu.async_copy(
            x_hbm.at[idx_v.at[pl.ds(0, _GCHUNK)]], rows0, sem0)

        def body(m, carry):
            for par in range(2):
                ch = 2 * m + par
                nxt = ch + 1

                @pl.when(nxt < _GNCH)
                def _():
                    pltpu.async_copy(
                        x_hbm.at[idx_v.at[pl.ds(nxt * _GCHUNK, _GCHUNK)]],
                        bufs[1 - par], sems[1 - par])

                pltpu.make_async_copy(
                    x_hbm.at[idx_v.at[pl.ds(ch * _GCHUNK, _GCHUNK)]],
                    bufs[par], sems[par]).wait()
                pltpu.sync_copy(
                    bufs[par],
                    g_hbm.at[pl.ds(base + ch * _GCHUNK, _GCHUNK)])
            return carry

        lax.fori_loop(0, _GNCH // 2, body, 0)


def _gather(x, idx_buf, counts):
    mesh = plsc.VectorSubcoreMesh(core_axis_name="c", subcore_axis_name="s")
    f = pl.kernel(
        _gather_body,
        mesh=mesh,
        out_type=jax.ShapeDtypeStruct((E * C, D // 2), jnp.int32),
        scratch_types=[
            pltpu.VMEM((_GROWS,), jnp.int32),
            pltpu.VMEM((L,), jnp.int32),
            pltpu.VMEM((_GCHUNK, D // 2), jnp.int32),
            pltpu.VMEM((_GCHUNK, D // 2), jnp.int32),
            pltpu.SemaphoreType.DMA,
            pltpu.SemaphoreType.DMA,
        ],
    )
    return f(x, idx_buf, counts)


def _gemm_body(cnt_ref, g_ref, w13_ref, w2_ref, ws_ref, out_ref):
    cb = pl.program_id(1)
    active = cb * BC < cnt_ref[pl.program_id(0)]

    @pl.when(active)
    def _():
        h = lax.dot_general(g_ref[...].astype(jnp.float32), w13_ref[0],
                            (((1,), (1,)), ((), ())),
                            preferred_element_type=jnp.float32)
        g = h[:, :F]
        u = h[:, F:]
        act = (g * jax.nn.sigmoid(g)) * u
        o = lax.dot_general(act, w2_ref[0],
                            (((1,), (1,)), ((), ())),
                            preferred_element_type=jnp.float32)
        out_ref[...] = o * ws_ref[...]

    @pl.when(jnp.logical_not(active))
    def _():
        out_ref[...] = jnp.zeros_like(out_ref)


def _expert_gemms(counts, gathered, w13, w2, w_slot):
    grid_spec = pltpu.PrefetchScalarGridSpec(
        num_scalar_prefetch=1,
        grid=(E, NCB),
        in_specs=[
            pl.BlockSpec((BC, D), lambda e, cb, cnt: (
                jnp.where(cb * BC < cnt[e], e * NCB + cb, 0), 0)),
            pl.BlockSpec((1, 2 * F, D), lambda e, cb, cnt: (e, 0, 0)),
            pl.BlockSpec((1, D, F), lambda e, cb, cnt: (e, 0, 0)),
            pl.BlockSpec((BC, 1), lambda e, cb, cnt: (e * NCB + cb, 0)),
        ],
        out_specs=pl.BlockSpec((BC, D), lambda e, cb, cnt: (
            jnp.where(cb * BC < cnt[e], e * NCB + cb, E * NCB), 0)),
    )
    return pl.pallas_call(
        _gemm_body,
        grid_spec=grid_spec,
        out_shape=jax.ShapeDtypeStruct((E * C + BC, D), jnp.float32),
        compiler_params=pltpu.CompilerParams(
            dimension_semantics=("arbitrary", "arbitrary"),
            vmem_limit_bytes=100 * 1024 * 1024,
        ),
    )(counts, gathered, w13, w2, w_slot)


_CTOK = T // NW
_CCH = 8


def _combine_body(oute_hbm, sp_hbm, out_hbm, sp_v, rows_v, out_v, sem):
    wid = lax.axis_index("s") * NC + lax.axis_index("c")
    tbase = wid * _CTOK
    pltpu.sync_copy(sp_hbm.at[pl.ds(tbase * K, _CTOK * K)], sp_v)

    def chunk(ch, carry):
        pltpu.async_copy(
            oute_hbm.at[sp_v.at[pl.ds(ch * _CCH * K, _CCH * K)]],
            rows_v, sem).wait()

        def lane_body(lb, carry2):
            col = lb * L
            for i in range(_CCH):
                out_v[i, pl.ds(col, L)] = (
                    rows_v[2 * i, pl.ds(col, L)]
                    + rows_v[2 * i + 1, pl.ds(col, L)])
            return carry2

        lax.fori_loop(0, D // L, lane_body, 0)
        pltpu.sync_copy(out_v, out_hbm.at[pl.ds(tbase + ch * _CCH, _CCH)])
        return carry

    lax.fori_loop(0, _CTOK // _CCH, chunk, 0)


def _combine(out_e, slot_pair):
    mesh = plsc.VectorSubcoreMesh(core_axis_name="c", subcore_axis_name="s")
    f = pl.kernel(
        _combine_body,
        mesh=mesh,
        out_type=jax.ShapeDtypeStruct((T, D), jnp.float32),
        scratch_types=[
            pltpu.VMEM((_CTOK * K,), jnp.int32),
            pltpu.VMEM((_CCH * K, D), jnp.float32),
            pltpu.VMEM((_CCH, D), jnp.float32),
            pltpu.SemaphoreType.DMA,
        ],
    )
    return f(out_e, slot_pair)


@jax.jit
def kernel(x, router_w, w13, w2):
    topi, topw = _routing(x, router_w)
    idx_buf, w_slot, slot_pair, counts = _dispatch(topi, topw)
    xi = lax.bitcast_convert_type(
        x.astype(jnp.bfloat16).reshape(T, D // 2, 2), jnp.int32)
    g_i = _gather(xi, idx_buf[:E * C], counts)
    gathered = lax.bitcast_convert_type(g_i, jnp.bfloat16).reshape(E * C, D)
    out_e = _expert_gemms(counts, gathered, w13, w2,
                          w_slot[:E * C].reshape(E * C, 1))
    return _combine(out_e, slot_pair)


# speedup vs baseline: 1.6921x; 1.6921x over previous
import jax
import jax.numpy as jnp
from jax import lax
from jax.experimental import pallas as pl
from jax.experimental.pallas import tpu as pltpu
from jax.experimental.pallas import tpu_sc as plsc

T = 2048
D = 2048
F = 1024
E = 8
K = 2
C = 2 * T * K // E

NC, NS, L = 2, 16, 16
NW = NC * NS

BT = 256
BC = 256
NCB = C // BC

_STAGE = 128
_DUMP = E * C


def _routing_body(x_ref, rw_ref, topi_ref, topw_ref):
    lt = lax.dot_general(rw_ref[...], x_ref[...],
                         (((0,), (1,)), ((), ())),
                         preferred_element_type=jnp.float32)
    row = lax.broadcasted_iota(jnp.int32, (E, BT), 0)
    m1 = jnp.max(lt, axis=0, keepdims=True)
    i1 = jnp.min(jnp.where(lt == m1, row, E), axis=0, keepdims=True)
    l2 = jnp.where(row == i1, -3e38, lt)
    m2 = jnp.max(l2, axis=0, keepdims=True)
    i2 = jnp.min(jnp.where(l2 == m2, row, E), axis=0, keepdims=True)
    r = jnp.exp(m2 - m1)
    w1 = 1.0 / (1.0 + r)
    w2 = r / (1.0 + r)
    topi_ref[...] = jnp.where(row == 0, i1, jnp.where(row == 1, i2, 0))
    topw_ref[...] = jnp.where(row == 0, w1, jnp.where(row == 1, w2, 0.0))


def _routing(x, router_w):
    return pl.pallas_call(
        _routing_body,
        grid=(T // BT,),
        in_specs=[
            pl.BlockSpec((BT, D), lambda i: (i, 0)),
            pl.BlockSpec((D, E), lambda i: (0, 0)),
        ],
        out_specs=[
            pl.BlockSpec((E, BT), lambda i: (0, i)),
            pl.BlockSpec((E, BT), lambda i: (0, i)),
        ],
        out_shape=[
            jax.ShapeDtypeStruct((E, T), jnp.int32),
            jax.ShapeDtypeStruct((E, T), jnp.float32),
        ],
    )(x, router_w)


def _dg(v, idx):
    return lax.gather(
        v, idx[:, None],
        lax.GatherDimensionNumbers(
            offset_dims=(), collapsed_slice_dims=(0,), start_index_map=(0,)),
        (1,), mode=lax.GatherScatterMode.PROMISE_IN_BOUNDS)


def _dispatch_body(topi_hbm, topw_hbm, idx_hbm, wslot_hbm, spair_hbm,
                   cnt_hbm, topi_v, topw_v, spair_v, zi_v, zf_v,
                   slot_st, tok_st, w_st, outv_v, sem):
    wid = lax.axis_index("s") * NC + lax.axis_index("c")

    @pl.when(wid == 0)
    def _():
        pltpu.sync_copy(topi_hbm.at[pl.ds(0, K)], topi_v)
        pltpu.sync_copy(topw_hbm.at[pl.ds(0, K)], topw_v)

        lanes = jnp.arange(L, dtype=jnp.int32)
        half = lanes >> 1
        parity_odd = (lanes & 1) == 1
        shift_idx = [jnp.maximum(lanes - d, 0) for d in (1, 2, 4, 8)]
        shift_ok = [lanes >= d for d in (1, 2, 4, 8)]
        bfly_idx = [lanes ^ d for d in (1, 2, 4, 8)]
        last_idx = jnp.full((L,), L - 1, jnp.int32)

        def zinit(i, c):
            zi_v[pl.ds(i * L, L)] = jnp.zeros((L,), jnp.int32)
            zf_v[pl.ds(i * L, L)] = jnp.zeros((L,), jnp.float32)
            return c

        lax.fori_loop(0, C // L, zinit, 0)

        def zout(i, c):
            pltpu.sync_copy(zi_v, idx_hbm.at[pl.ds(i * C, C)])
            pltpu.sync_copy(zf_v, wslot_hbm.at[pl.ds(i * C, C)])
            return c

        lax.fori_loop(0, E, zout, 0)

        def outer(m, cnt):
            ea = topi_v[0, pl.ds(m * L, L)]
            eb = topi_v[1, pl.ds(m * L, L)]
            wa = topw_v[0, pl.ds(m * L, L)]
            wb = topw_v[1, pl.ds(m * L, L)]
            for h in range(2):
                p = half + (h * (L // 2))
                e_vec = jnp.where(parity_odd, _dg(eb, p), _dg(ea, p))
                w_vec = jnp.where(parity_odd, _dg(wb, p), _dg(wa, p))
                t_vec = m * L + h * (L // 2) + half
                base = _dg(cnt, e_vec)
                pos = jnp.zeros((L,), jnp.int32)
                for x in range(E):
                    mx = e_vec == x
                    s = jnp.where(mx, 1, 0)
                    for d in range(4):
                        s = s + jnp.where(shift_ok[d], _dg(s, shift_idx[d]), 0)
                    pos = jnp.where(mx, s - 1, pos)
                    cnt = cnt + jnp.where(lanes == x, _dg(s, last_idx), 0)
                pos = base + pos
                valid = pos < C
                slot = e_vec * C + pos
                q = (m & 3) * (2 * L) + h * L
                slot_st[pl.ds(q, L)] = jnp.where(valid, slot, _DUMP + lanes)
                tok_st[pl.ds(q, L)] = t_vec
                w_st[pl.ds(q, L)] = w_vec
                spair_v[pl.ds(m * (2 * L) + h * L, L)] = (
                    jnp.where(valid, slot, -1))

            @pl.when((m & 3) == 3)
            def _():
                pltpu.async_copy(tok_st, idx_hbm.at[slot_st], sem).wait()
                pltpu.async_copy(w_st, wslot_hbm.at[slot_st], sem).wait()

            return cnt

        cnt = lax.fori_loop(0, T // L, outer,
                            jnp.zeros((L,), jnp.int32))

        cv = jnp.minimum(cnt, C)
        outv_v[...] = cv
        pltpu.sync_copy(outv_v, cnt_hbm)

        def fix(j, c):
            sp = spair_v[pl.ds(j * L, L)]
            spair_v[pl.ds(j * L, L)] = jnp.where(sp < 0, E * C, sp)
            return c

        lax.fori_loop(0, (T * K) // L, fix, 0)
        pltpu.sync_copy(spair_v, spair_hbm)


def _dispatch(topi, topw):
    mesh = plsc.VectorSubcoreMesh(core_axis_name="c", subcore_axis_name="s")
    f = pl.kernel(
        _dispatch_body,
        mesh=mesh,
        out_type=[
            jax.ShapeDtypeStruct((E * C + L,), jnp.int32),
            jax.ShapeDtypeStruct((E * C + L,), jnp.float32),
            jax.ShapeDtypeStruct((T * K,), jnp.int32),
            jax.ShapeDtypeStruct((L,), jnp.int32),
        ],
        scratch_types=[
            pltpu.VMEM((K, T), jnp.int32),
            pltpu.VMEM((K, T), jnp.float32),
            pltpu.VMEM((T * K,), jnp.int32),
            pltpu.VMEM((C,), jnp.int32),
            pltpu.VMEM((C,), jnp.float32),
            pltpu.VMEM((_STAGE,), jnp.int32),
            pltpu.VMEM((_STAGE,), jnp.int32),
            pltpu.VMEM((_STAGE,), jnp.float32),
            pltpu.VMEM((L,), jnp.int32),
            pltpu.SemaphoreType.DMA,
        ],
    )
    return f(topi, topw)


_GROWS = (E * C) // NW
_GCHUNK = 16
_GNCH = _GROWS // _GCHUNK


def _gather_body(x_hbm, idx_hbm, cnt_hbm, g_hbm, idx_v, cv_v,
                 rows0, rows1, sem0, sem1):
    wid = lax.axis_index("s") * NC + lax.axis_index("c")
    pltpu.sync_copy(cnt_hbm, cv_v)
    myexp = wid >> 2
    jrot = ((wid & 3) + myexp) & 3
    base = myexp * C + jrot * _GROWS
    mybase = jrot * _GROWS
    cv = cv_v[pl.ds(0, L)]
    my_cnt = cv[0]
    for i in range(1, E):
        my_cnt = jnp.where(myexp == i, cv[i], my_cnt)
    active = my_cnt > mybase

    @pl.when(active)
    def _():
        pltpu.sync_copy(idx_hbm.at[pl.ds(base, _GROWS)], idx_v)
        bufs = (rows0, rows1)
        sems = (sem0, sem1)
        pltpu.async_copy(
            x_hbm.at[idx_v.at[pl.ds(0, _GCHUNK)]], rows0, sem0)

        def body(m, carry):
            for par in range(2):
                ch = 2 * m + par
                nxt = ch + 1

                @pl.when(nxt < _GNCH)
                def _():
                    pltpu.async_copy(
                        x_hbm.at[idx_v.at[pl.ds(nxt * _GCHUNK, _GCHUNK)]],
                        bufs[1 - par], sems[1 - par])

                pltpu.make_async_copy(
                    x_hbm.at[idx_v.at[pl.ds(ch * _GCHUNK, _GCHUNK)]],
                    bufs[par], sems[par]).wait()
                pltpu.sync_copy(
                    bufs[par],
                    g_hbm.at[pl.ds(base + ch * _GCHUNK, _GCHUNK)])
            return carry

        lax.fori_loop(0, _GNCH // 2, body, 0)


def _gather(x, idx_buf, counts):
    mesh = plsc.VectorSubcoreMesh(core_axis_name="c", subcore_axis_name="s")
    f = pl.kernel(
        _gather_body,
        mesh=mesh,
        out_type=jax.ShapeDtypeStruct((E * C, D), jnp.float32),
        scratch_types=[
            pltpu.VMEM((_GROWS,), jnp.int32),
            pltpu.VMEM((L,), jnp.int32),
            pltpu.VMEM((_GCHUNK, D), jnp.float32),
            pltpu.VMEM((_GCHUNK, D), jnp.float32),
            pltpu.SemaphoreType.DMA,
            pltpu.SemaphoreType.DMA,
        ],
    )
    return f(x, idx_buf, counts)


def _gemm_body(cnt_ref, g_ref, w13_ref, w2_ref, ws_ref, out_ref):
    cb = pl.program_id(1)
    active = cb * BC < cnt_ref[pl.program_id(0)]

    @pl.when(active)
    def _():
        h = lax.dot_general(g_ref[...], w13_ref[0],
                            (((1,), (1,)), ((), ())),
                            preferred_element_type=jnp.float32)
        g = h[:, :F]
        u = h[:, F:]
        act = (g * jax.nn.sigmoid(g)) * u
        o = lax.dot_general(act, w2_ref[0],
                            (((1,), (1,)), ((), ())),
                            preferred_element_type=jnp.float32)
        out_ref[...] = o * ws_ref[...]

    @pl.when(jnp.logical_not(active))
    def _():
        out_ref[...] = jnp.zeros_like(out_ref)


def _expert_gemms(counts, gathered, w13, w2, w_slot):
    grid_spec = pltpu.PrefetchScalarGridSpec(
        num_scalar_prefetch=1,
        grid=(E, NCB),
        in_specs=[
            pl.BlockSpec((BC, D), lambda e, cb, cnt: (
                jnp.where(cb * BC < cnt[e], e * NCB + cb, 0), 0)),
            pl.BlockSpec((1, 2 * F, D), lambda e, cb, cnt: (e, 0, 0)),
            pl.BlockSpec((1, D, F), lambda e, cb, cnt: (e, 0, 0)),
            pl.BlockSpec((BC, 1), lambda e, cb, cnt: (e * NCB + cb, 0)),
        ],
        out_specs=pl.BlockSpec((BC, D), lambda e, cb, cnt: (
            jnp.where(cb * BC < cnt[e], e * NCB + cb, E * NCB), 0)),
    )
    return pl.pallas_call(
        _gemm_body,
        grid_spec=grid_spec,
        out_shape=jax.ShapeDtypeStruct((E * C + BC, D), jnp.float32),
        compiler_params=pltpu.CompilerParams(
            dimension_semantics=("arbitrary", "arbitrary"),
            vmem_limit_bytes=100 * 1024 * 1024,
        ),
    )(counts, gathered, w13, w2, w_slot)


_CTOK = T // NW
_CCH = 8


def _combine_body(oute_hbm, sp_hbm, out_hbm, sp_v, rows_v, out_v, sem):
    wid = lax.axis_index("s") * NC + lax.axis_index("c")
    tbase = wid * _CTOK
    pltpu.sync_copy(sp_hbm.at[pl.ds(tbase * K, _CTOK * K)], sp_v)

    def chunk(ch, carry):
        pltpu.async_copy(
            oute_hbm.at[sp_v.at[pl.ds(ch * _CCH * K, _CCH * K)]],
            rows_v, sem).wait()

        def lane_body(lb, carry2):
            col = lb * L
            for i in range(_CCH):
                out_v[i, pl.ds(col, L)] = (
                    rows_v[2 * i, pl.ds(col, L)]
                    + rows_v[2 * i + 1, pl.ds(col, L)])
            return carry2

        lax.fori_loop(0, D // L, lane_body, 0)
        pltpu.sync_copy(out_v, out_hbm.at[pl.ds(tbase + ch * _CCH, _CCH)])
        return carry

    lax.fori_loop(0, _CTOK // _CCH, chunk, 0)


def _combine(out_e, slot_pair):
    mesh = plsc.VectorSubcoreMesh(core_axis_name="c", subcore_axis_name="s")
    f = pl.kernel(
        _combine_body,
        mesh=mesh,
        out_type=jax.ShapeDtypeStruct((T, D), jnp.float32),
        scratch_types=[
            pltpu.VMEM((_CTOK * K,), jnp.int32),
            pltpu.VMEM((_CCH * K, D), jnp.float32),
            pltpu.VMEM((_CCH, D), jnp.float32),
            pltpu.SemaphoreType.DMA,
        ],
    )
    return f(out_e, slot_pair)


@jax.jit
def kernel(x, router_w, w13, w2):
    topi, topw = _routing(x, router_w)
    idx_buf, w_slot, slot_pair, counts = _dispatch(topi, topw)
    gathered = _gather(x, idx_buf[:E * C], counts)
    out_e = _expert_gemms(counts, gathered, w13, w2,
                          w_slot[:E * C].reshape(E * C, 1))
    return _combine(out_e, slot_pair)


# speedup vs baseline: 1.7542x; 1.0367x over previous
import jax
import jax.numpy as jnp
from jax import lax
from jax.experimental import pallas as pl
from jax.experimental.pallas import tpu as pltpu
from jax.experimental.pallas import tpu_sc as plsc

T = 2048
D = 2048
F = 1024
E = 8
K = 2
C = 2 * T * K // E

NC, NS, L = 2, 16, 16
NW = NC * NS

BT = 256
BC = 256
NCB = C // BC

_STAGE = 128
_DUMP = E * C


def _routing_body(x_ref, rw_ref, topi_ref, topw_ref):
    lt = lax.dot_general(rw_ref[...], x_ref[...],
                         (((0,), (1,)), ((), ())),
                         preferred_element_type=jnp.float32)
    row = lax.broadcasted_iota(jnp.int32, (E, BT), 0)
    m1 = jnp.max(lt, axis=0, keepdims=True)
    i1 = jnp.min(jnp.where(lt == m1, row, E), axis=0, keepdims=True)
    l2 = jnp.where(row == i1, -3e38, lt)
    m2 = jnp.max(l2, axis=0, keepdims=True)
    i2 = jnp.min(jnp.where(l2 == m2, row, E), axis=0, keepdims=True)
    r = jnp.exp(m2 - m1)
    w1 = 1.0 / (1.0 + r)
    w2 = r / (1.0 + r)
    topi_ref[...] = jnp.where(row == 0, i1, jnp.where(row == 1, i2, 0))
    topw_ref[...] = jnp.where(row == 0, w1, jnp.where(row == 1, w2, 0.0))


def _routing(x, router_w):
    return pl.pallas_call(
        _routing_body,
        grid=(T // BT,),
        in_specs=[
            pl.BlockSpec((BT, D), lambda i: (i, 0)),
            pl.BlockSpec((D, E), lambda i: (0, 0)),
        ],
        out_specs=[
            pl.BlockSpec((E, BT), lambda i: (0, i)),
            pl.BlockSpec((E, BT), lambda i: (0, i)),
        ],
        out_shape=[
            jax.ShapeDtypeStruct((E, T), jnp.int32),
            jax.ShapeDtypeStruct((E, T), jnp.float32),
        ],
    )(x, router_w)


def _dg(v, idx):
    return lax.gather(
        v, idx[:, None],
        lax.GatherDimensionNumbers(
            offset_dims=(), collapsed_slice_dims=(0,), start_index_map=(0,)),
        (1,), mode=lax.GatherScatterMode.PROMISE_IN_BOUNDS)


def _dispatch_body(topi_hbm, topw_hbm, idx_hbm, wslot_hbm, spair_hbm,
                   cnt_hbm, topi_v, topw_v, spair_v, zi_v, zf_v,
                   slot_st, tok_st, w_st, outv_v, sem):
    wid = lax.axis_index("s") * NC + lax.axis_index("c")

    @pl.when(wid == 0)
    def _():
        pltpu.sync_copy(topi_hbm.at[pl.ds(0, K)], topi_v)
        pltpu.sync_copy(topw_hbm.at[pl.ds(0, K)], topw_v)

        lanes = jnp.arange(L, dtype=jnp.int32)
        half = lanes >> 1
        parity_odd = (lanes & 1) == 1
        shift_idx = [jnp.maximum(lanes - d, 0) for d in (1, 2, 4, 8)]
        shift_ok = [lanes >= d for d in (1, 2, 4, 8)]
        bfly_idx = [lanes ^ d for d in (1, 2, 4, 8)]
        last_idx = jnp.full((L,), L - 1, jnp.int32)

        def zinit(i, c):
            zi_v[pl.ds(i * L, L)] = jnp.zeros((L,), jnp.int32)
            zf_v[pl.ds(i * L, L)] = jnp.zeros((L,), jnp.float32)
            return c

        lax.fori_loop(0, C // L, zinit, 0)

        def zout(i, c):
            pltpu.sync_copy(zi_v, idx_hbm.at[pl.ds(i * C, C)])
            pltpu.sync_copy(zf_v, wslot_hbm.at[pl.ds(i * C, C)])
            return c

        lax.fori_loop(0, E, zout, 0)

        def outer(m, cnt):
            ea = topi_v[0, pl.ds(m * L, L)]
            eb = topi_v[1, pl.ds(m * L, L)]
            wa = topw_v[0, pl.ds(m * L, L)]
            wb = topw_v[1, pl.ds(m * L, L)]
            for h in range(2):
                p = half + (h * (L // 2))
                e_vec = jnp.where(parity_odd, _dg(eb, p), _dg(ea, p))
                w_vec = jnp.where(parity_odd, _dg(wb, p), _dg(wa, p))
                t_vec = m * L + h * (L // 2) + half
                base = _dg(cnt, e_vec)
                pos = jnp.zeros((L,), jnp.int32)
                for x in range(E):
                    mx = e_vec == x
                    s = jnp.where(mx, 1, 0)
                    for d in range(4):
                        s = s + jnp.where(shift_ok[d], _dg(s, shift_idx[d]), 0)
                    pos = jnp.where(mx, s - 1, pos)
                    cnt = cnt + jnp.where(lanes == x, _dg(s, last_idx), 0)
                pos = base + pos
                valid = pos < C
                slot = e_vec * C + pos
                q = (m & 3) * (2 * L) + h * L
                slot_st[pl.ds(q, L)] = jnp.where(valid, slot, _DUMP + lanes)
                tok_st[pl.ds(q, L)] = t_vec
                w_st[pl.ds(q, L)] = w_vec
                spair_v[pl.ds(m * (2 * L) + h * L, L)] = (
                    jnp.where(valid, slot, -1))

            @pl.when((m & 3) == 3)
            def _():
                pltpu.async_copy(tok_st, idx_hbm.at[slot_st], sem).wait()
                pltpu.async_copy(w_st, wslot_hbm.at[slot_st], sem).wait()

            return cnt

        cnt = lax.fori_loop(0, T // L, outer,
                            jnp.zeros((L,), jnp.int32))

        cv = jnp.minimum(cnt, C)
        outv_v[...] = cv
        pltpu.sync_copy(outv_v, cnt_hbm)

        def fix(j, c):
            sp = spair_v[pl.ds(j * L, L)]
            spair_v[pl.ds(j * L, L)] = jnp.where(sp < 0, E * C, sp)
            return c

        lax.fori_loop(0, (T * K) // L, fix, 0)
        pltpu.sync_copy(spair_v, spair_hbm)


def _dispatch(topi, topw):
    mesh = plsc.VectorSubcoreMesh(core_axis_name="c", subcore_axis_name="s")
    f = pl.kernel(
        _dispatch_body,
        mesh=mesh,
        out_type=[
            jax.ShapeDtypeStruct((E * C + L,), jnp.int32),
            jax.ShapeDtypeStruct((E * C + L,), jnp.float32),
            jax.ShapeDtypeStruct((T * K,), jnp.int32),
            jax.ShapeDtypeStruct((L,), jnp.int32),
        ],
        scratch_types=[
            pltpu.VMEM((K, T), jnp.int32),
            pltpu.VMEM((K, T), jnp.float32),
            pltpu.VMEM((T * K,), jnp.int32),
            pltpu.VMEM((C,), jnp.int32),
            pltpu.VMEM((C,), jnp.float32),
            pltpu.VMEM((_STAGE,), jnp.int32),
            pltpu.VMEM((_STAGE,), jnp.int32),
            pltpu.VMEM((_STAGE,), jnp.float32),
            pltpu.VMEM((L,), jnp.int32),
            pltpu.SemaphoreType.DMA,
        ],
    )
    return f(topi, topw)


_GROWS = (E * C) // NW
_GCHUNK = 16
_GNCH = _GROWS // _GCHUNK


def _gather_body(x_hbm, idx_hbm, cnt_hbm, g_hbm, idx_v, cv_v,
                 rows0, rows1, gsem0, gsem1, osem0, osem1):
    wid = lax.axis_index("s") * NC + lax.axis_index("c")
    pltpu.sync_copy(cnt_hbm, cv_v)
    myexp = wid >> 2
    jrot = ((wid & 3) + myexp) & 3
    base = myexp * C + jrot * _GROWS
    mybase = jrot * _GROWS
    cv = cv_v[pl.ds(0, L)]
    my_cnt = cv[0]
    for i in range(1, E):
        my_cnt = jnp.where(myexp == i, cv[i], my_cnt)
    active = my_cnt > mybase

    @pl.when(active)
    def _():
        pltpu.sync_copy(idx_hbm.at[pl.ds(base, _GROWS)], idx_v)
        bufs = (rows0, rows1)
        gsems = (gsem0, gsem1)
        osems = (osem0, osem1)
        pltpu.async_copy(
            x_hbm.at[idx_v.at[pl.ds(0, _GCHUNK)]], rows0, gsem0)

        def body(m, carry):
            for par in range(2):
                ch = 2 * m + par
                pltpu.make_async_copy(
                    x_hbm.at[idx_v.at[pl.ds(ch * _GCHUNK, _GCHUNK)]],
                    bufs[par], gsems[par]).wait()
                pltpu.async_copy(
                    bufs[par],
                    g_hbm.at[pl.ds(base + ch * _GCHUNK, _GCHUNK)],
                    osems[par])

                @pl.when(ch >= 1)
                def _():
                    pltpu.make_async_copy(
                        bufs[1 - par],
                        g_hbm.at[pl.ds(base + (ch - 1) * _GCHUNK, _GCHUNK)],
                        osems[1 - par]).wait()

                @pl.when(ch + 1 < _GNCH)
                def _():
                    pltpu.async_copy(
                        x_hbm.at[idx_v.at[pl.ds((ch + 1) * _GCHUNK,
                                                _GCHUNK)]],
                        bufs[1 - par], gsems[1 - par])
            return carry

        lax.fori_loop(0, _GNCH // 2, body, 0)
        pltpu.make_async_copy(
            bufs[1],
            g_hbm.at[pl.ds(base + (_GNCH - 1) * _GCHUNK, _GCHUNK)],
            osems[1]).wait()


def _gather(x, idx_buf, counts):
    mesh = plsc.VectorSubcoreMesh(core_axis_name="c", subcore_axis_name="s")
    f = pl.kernel(
        _gather_body,
        mesh=mesh,
        out_type=jax.ShapeDtypeStruct((E * C, D), jnp.float32),
        scratch_types=[
            pltpu.VMEM((_GROWS,), jnp.int32),
            pltpu.VMEM((L,), jnp.int32),
            pltpu.VMEM((_GCHUNK, D), jnp.float32),
            pltpu.VMEM((_GCHUNK, D), jnp.float32),
            pltpu.SemaphoreType.DMA,
            pltpu.SemaphoreType.DMA,
            pltpu.SemaphoreType.DMA,
            pltpu.SemaphoreType.DMA,
        ],
    )
    return f(x, idx_buf, counts)


def _gemm_body(cnt_ref, g_ref, w13_ref, w2_ref, ws_ref, out_ref):
    cb = pl.program_id(1)
    active = cb * BC < cnt_ref[pl.program_id(0)]

    @pl.when(active)
    def _():
        h = lax.dot_general(g_ref[...], w13_ref[0],
                            (((1,), (1,)), ((), ())),
                            preferred_element_type=jnp.float32)
        g = h[:, :F]
        u = h[:, F:]
        act = (g * jax.nn.sigmoid(g)) * u
        o = lax.dot_general(act, w2_ref[0],
                            (((1,), (1,)), ((), ())),
                            preferred_element_type=jnp.float32)
        out_ref[...] = o * ws_ref[...]

    @pl.when(jnp.logical_not(active))
    def _():
        out_ref[...] = jnp.zeros_like(out_ref)


def _expert_gemms(counts, gathered, w13, w2, w_slot):
    grid_spec = pltpu.PrefetchScalarGridSpec(
        num_scalar_prefetch=1,
        grid=(E, NCB),
        in_specs=[
            pl.BlockSpec((BC, D), lambda e, cb, cnt: (
                jnp.where(cb * BC < cnt[e], e * NCB + cb, 0), 0)),
            pl.BlockSpec((1, 2 * F, D), lambda e, cb, cnt: (e, 0, 0)),
            pl.BlockSpec((1, D, F), lambda e, cb, cnt: (e, 0, 0)),
            pl.BlockSpec((BC, 1), lambda e, cb, cnt: (e * NCB + cb, 0)),
        ],
        out_specs=pl.BlockSpec((BC, D), lambda e, cb, cnt: (
            jnp.where(cb * BC < cnt[e], e * NCB + cb, E * NCB), 0)),
    )
    return pl.pallas_call(
        _gemm_body,
        grid_spec=grid_spec,
        out_shape=jax.ShapeDtypeStruct((E * C + BC, D), jnp.float32),
        compiler_params=pltpu.CompilerParams(
            dimension_semantics=("arbitrary", "arbitrary"),
            vmem_limit_bytes=100 * 1024 * 1024,
        ),
    )(counts, gathered, w13, w2, w_slot)


_CTOK = T // NW
_CCH = 8


def _combine_body(oute_hbm, sp_hbm, out_hbm, sp_v, rows0, rows1, out_v,
                  sem0, sem1):
    wid = lax.axis_index("s") * NC + lax.axis_index("c")
    tbase = wid * _CTOK
    pltpu.sync_copy(sp_hbm.at[pl.ds(tbase * K, _CTOK * K)], sp_v)
    bufs = (rows0, rows1)
    sems = (sem0, sem1)
    nch = _CTOK // _CCH
    pltpu.async_copy(
        oute_hbm.at[sp_v.at[pl.ds(0, _CCH * K)]], rows0, sem0)

    def chunk(m, carry):
        for par in range(2):
            ch = 2 * m + par

            @pl.when(ch + 1 < nch)
            def _():
                pltpu.async_copy(
                    oute_hbm.at[sp_v.at[pl.ds((ch + 1) * _CCH * K,
                                              _CCH * K)]],
                    bufs[1 - par], sems[1 - par])

            pltpu.make_async_copy(
                oute_hbm.at[sp_v.at[pl.ds(ch * _CCH * K, _CCH * K)]],
                bufs[par], sems[par]).wait()
            rows_v = bufs[par]

            def lane_body(lb, carry2):
                col = lb * L
                for i in range(_CCH):
                    out_v[i, pl.ds(col, L)] = (
                        rows_v[2 * i, pl.ds(col, L)]
                        + rows_v[2 * i + 1, pl.ds(col, L)])
                return carry2

            lax.fori_loop(0, D // L, lane_body, 0)
            pltpu.sync_copy(out_v,
                            out_hbm.at[pl.ds(tbase + ch * _CCH, _CCH)])
        return carry

    lax.fori_loop(0, _CTOK // _CCH // 2, chunk, 0)


def _combine(out_e, slot_pair):
    mesh = plsc.VectorSubcoreMesh(core_axis_name="c", subcore_axis_name="s")
    f = pl.kernel(
        _combine_body,
        mesh=mesh,
        out_type=jax.ShapeDtypeStruct((T, D), jnp.float32),
        scratch_types=[
            pltpu.VMEM((_CTOK * K,), jnp.int32),
            pltpu.VMEM((_CCH * K, D), jnp.float32),
            pltpu.VMEM((_CCH * K, D), jnp.float32),
            pltpu.VMEM((_CCH, D), jnp.float32),
            pltpu.SemaphoreType.DMA,
            pltpu.SemaphoreType.DMA,
        ],
    )
    return f(out_e, slot_pair)


@jax.jit
def kernel(x, router_w, w13, w2):
    topi, topw = _routing(x, router_w)
    idx_buf, w_slot, slot_pair, counts = _dispatch(topi, topw)
    gathered = _gather(x, idx_buf[:E * C], counts)
    out_e = _expert_gemms(counts, gathered, w13, w2,
                          w_slot[:E * C].reshape(E * C, 1))
    return _combine(out_e, slot_pair)


# speedup vs baseline: 1.7547x; 1.0003x over previous
import jax
import jax.numpy as jnp
from jax import lax
from jax.experimental import pallas as pl
from jax.experimental.pallas import tpu as pltpu
from jax.experimental.pallas import tpu_sc as plsc

T = 2048
D = 2048
F = 1024
E = 8
K = 2
C = 2 * T * K // E

NC, NS, L = 2, 16, 16
NW = NC * NS

BT = 256
BC = 256
NCB = C // BC

_STAGE = 128
_DUMP = E * C


def _routing_body(x_ref, rw_ref, topi_ref, topw_ref):
    lt = lax.dot_general(rw_ref[...], x_ref[...],
                         (((0,), (1,)), ((), ())),
                         preferred_element_type=jnp.float32)
    row = lax.broadcasted_iota(jnp.int32, (E, BT), 0)
    m1 = jnp.max(lt, axis=0, keepdims=True)
    i1 = jnp.min(jnp.where(lt == m1, row, E), axis=0, keepdims=True)
    l2 = jnp.where(row == i1, -3e38, lt)
    m2 = jnp.max(l2, axis=0, keepdims=True)
    i2 = jnp.min(jnp.where(l2 == m2, row, E), axis=0, keepdims=True)
    r = jnp.exp(m2 - m1)
    w1 = 1.0 / (1.0 + r)
    w2 = r / (1.0 + r)
    topi_ref[...] = jnp.where(row == 0, i1, jnp.where(row == 1, i2, 0))
    topw_ref[...] = jnp.where(row == 0, w1, jnp.where(row == 1, w2, 0.0))


def _routing(x, router_w):
    return pl.pallas_call(
        _routing_body,
        grid=(T // BT,),
        in_specs=[
            pl.BlockSpec((BT, D), lambda i: (i, 0)),
            pl.BlockSpec((D, E), lambda i: (0, 0)),
        ],
        out_specs=[
            pl.BlockSpec((E, BT), lambda i: (0, i)),
            pl.BlockSpec((E, BT), lambda i: (0, i)),
        ],
        out_shape=[
            jax.ShapeDtypeStruct((E, T), jnp.int32),
            jax.ShapeDtypeStruct((E, T), jnp.float32),
        ],
    )(x, router_w)


def _dg(v, idx):
    return lax.gather(
        v, idx[:, None],
        lax.GatherDimensionNumbers(
            offset_dims=(), collapsed_slice_dims=(0,), start_index_map=(0,)),
        (1,), mode=lax.GatherScatterMode.PROMISE_IN_BOUNDS)


def _dispatch_body(topi_hbm, topw_hbm, idx_hbm, wslot_hbm, spair_hbm,
                   cnt_hbm, topi_v, topw_v, spair_v, zi_v, zf_v,
                   slot_st, tok_st, w_st, outv_v, sem):
    wid = lax.axis_index("s") * NC + lax.axis_index("c")

    @pl.when(wid == 0)
    def _():
        pltpu.sync_copy(topi_hbm.at[pl.ds(0, K)], topi_v)
        pltpu.sync_copy(topw_hbm.at[pl.ds(0, K)], topw_v)

        lanes = jnp.arange(L, dtype=jnp.int32)
        half = lanes >> 1
        parity_odd = (lanes & 1) == 1
        shift_idx = [jnp.maximum(lanes - d, 0) for d in (1, 2, 4, 8)]
        shift_ok = [lanes >= d for d in (1, 2, 4, 8)]
        last_idx = jnp.full((L,), L - 1, jnp.int32)

        def zinit(i, c):
            zi_v[pl.ds(i * L, L)] = jnp.zeros((L,), jnp.int32)
            zf_v[pl.ds(i * L, L)] = jnp.zeros((L,), jnp.float32)
            return c

        lax.fori_loop(0, C // L, zinit, 0)

        def zout(i, c):
            pltpu.sync_copy(zi_v, idx_hbm.at[pl.ds(i * C, C)])
            pltpu.sync_copy(zf_v, wslot_hbm.at[pl.ds(i * C, C)])
            return c

        lax.fori_loop(0, E, zout, 0)

        def outer(m, cnt):
            ea = topi_v[0, pl.ds(m * L, L)]
            eb = topi_v[1, pl.ds(m * L, L)]
            wa = topw_v[0, pl.ds(m * L, L)]
            wb = topw_v[1, pl.ds(m * L, L)]
            for h in range(2):
                p = half + (h * (L // 2))
                e_vec = jnp.where(parity_odd, _dg(eb, p), _dg(ea, p))
                w_vec = jnp.where(parity_odd, _dg(wb, p), _dg(wa, p))
                t_vec = m * L + h * (L // 2) + half
                base = _dg(cnt, e_vec)
                pos = jnp.zeros((L,), jnp.int32)
                for x in range(E):
                    mx = e_vec == x
                    s = jnp.where(mx, 1, 0)
                    for d in range(4):
                        s = s + jnp.where(shift_ok[d], _dg(s, shift_idx[d]), 0)
                    pos = jnp.where(mx, s - 1, pos)
                    cnt = cnt + jnp.where(lanes == x, _dg(s, last_idx), 0)
                pos = base + pos
                valid = pos < C
                slot = e_vec * C + pos
                q = (m & 3) * (2 * L) + h * L
                slot_st[pl.ds(q, L)] = jnp.where(valid, slot, _DUMP + lanes)
                tok_st[pl.ds(q, L)] = t_vec
                w_st[pl.ds(q, L)] = w_vec
                spair_v[pl.ds(m * (2 * L) + h * L, L)] = (
                    jnp.where(valid, slot, -1))

            @pl.when((m & 3) == 3)
            def _():
                pltpu.async_copy(tok_st, idx_hbm.at[slot_st], sem).wait()
                pltpu.async_copy(w_st, wslot_hbm.at[slot_st], sem).wait()

            return cnt

        cnt = lax.fori_loop(0, T // L, outer,
                            jnp.zeros((L,), jnp.int32))

        cv = jnp.minimum(cnt, C)
        outv_v[...] = cv
        pltpu.sync_copy(outv_v, cnt_hbm)

        def fix(j, c):
            sp = spair_v[pl.ds(j * L, L)]
            spair_v[pl.ds(j * L, L)] = jnp.where(sp < 0, E * C, sp)
            return c

        lax.fori_loop(0, (T * K) // L, fix, 0)
        pltpu.sync_copy(spair_v, spair_hbm)


def _dispatch(topi, topw):
    mesh = plsc.VectorSubcoreMesh(core_axis_name="c", subcore_axis_name="s")
    f = pl.kernel(
        _dispatch_body,
        mesh=mesh,
        out_type=[
            jax.ShapeDtypeStruct((E * C + L,), jnp.int32),
            jax.ShapeDtypeStruct((E * C + L,), jnp.float32),
            jax.ShapeDtypeStruct((T * K,), jnp.int32),
            jax.ShapeDtypeStruct((L,), jnp.int32),
        ],
        scratch_types=[
            pltpu.VMEM((K, T), jnp.int32),
            pltpu.VMEM((K, T), jnp.float32),
            pltpu.VMEM((T * K,), jnp.int32),
            pltpu.VMEM((C,), jnp.int32),
            pltpu.VMEM((C,), jnp.float32),
            pltpu.VMEM((_STAGE,), jnp.int32),
            pltpu.VMEM((_STAGE,), jnp.int32),
            pltpu.VMEM((_STAGE,), jnp.float32),
            pltpu.VMEM((L,), jnp.int32),
            pltpu.SemaphoreType.DMA,
        ],
    )
    return f(topi, topw)


_GROWS = (E * C) // NW
_GCHUNK = 16
_GNCH = _GROWS // _GCHUNK


def _gather_body(x_hbm, idx_hbm, cnt_hbm, g_hbm, idx_v, cv_v,
                 rows0, rows1, gsem0, gsem1, osem0, osem1):
    wid = lax.axis_index("s") * NC + lax.axis_index("c")
    pltpu.sync_copy(cnt_hbm, cv_v)
    myexp = wid >> 2
    jrot = ((wid & 3) + myexp) & 3
    base = myexp * C + jrot * _GROWS
    mybase = jrot * _GROWS
    cv = cv_v[pl.ds(0, L)]
    my_cnt = cv[0]
    for i in range(1, E):
        my_cnt = jnp.where(myexp == i, cv[i], my_cnt)
    active = my_cnt > mybase

    @pl.when(active)
    def _():
        pltpu.sync_copy(idx_hbm.at[pl.ds(base, _GROWS)], idx_v)
        bufs = (rows0, rows1)
        gsems = (gsem0, gsem1)
        osems = (osem0, osem1)
        pltpu.async_copy(
            x_hbm.at[idx_v.at[pl.ds(0, _GCHUNK)]], rows0, gsem0)

        def body(m, carry):
            for par in range(2):
                ch = 2 * m + par
                pltpu.make_async_copy(
                    x_hbm.at[idx_v.at[pl.ds(ch * _GCHUNK, _GCHUNK)]],
                    bufs[par], gsems[par]).wait()
                pltpu.async_copy(
                    bufs[par],
                    g_hbm.at[pl.ds(base + ch * _GCHUNK, _GCHUNK)],
                    osems[par])

                @pl.when(ch >= 1)
                def _():
                    pltpu.make_async_copy(
                        bufs[1 - par],
                        g_hbm.at[pl.ds(base + (ch - 1) * _GCHUNK, _GCHUNK)],
                        osems[1 - par]).wait()

                @pl.when(ch + 1 < _GNCH)
                def _():
                    pltpu.async_copy(
                        x_hbm.at[idx_v.at[pl.ds((ch + 1) * _GCHUNK,
                                                _GCHUNK)]],
                        bufs[1 - par], gsems[1 - par])
            return carry

        lax.fori_loop(0, _GNCH // 2, body, 0)
        pltpu.make_async_copy(
            bufs[1],
            g_hbm.at[pl.ds(base + (_GNCH - 1) * _GCHUNK, _GCHUNK)],
            osems[1]).wait()


def _gather(x, idx_buf, counts):
    mesh = plsc.VectorSubcoreMesh(core_axis_name="c", subcore_axis_name="s")
    f = pl.kernel(
        _gather_body,
        mesh=mesh,
        out_type=jax.ShapeDtypeStruct((E * C, D), jnp.float32),
        scratch_types=[
            pltpu.VMEM((_GROWS,), jnp.int32),
            pltpu.VMEM((L,), jnp.int32),
            pltpu.VMEM((_GCHUNK, D), jnp.float32),
            pltpu.VMEM((_GCHUNK, D), jnp.float32),
            pltpu.SemaphoreType.DMA,
            pltpu.SemaphoreType.DMA,
            pltpu.SemaphoreType.DMA,
            pltpu.SemaphoreType.DMA,
        ],
    )
    return f(x, idx_buf, counts)


def _gemm_body(cnt_ref, g_ref, w13_ref, w2_ref, ws_ref, out_ref):
    cb = pl.program_id(1)
    active = cb * BC < cnt_ref[pl.program_id(0)]

    @pl.when(active)
    def _():
        h = lax.dot_general(g_ref[...], w13_ref[0],
                            (((1,), (1,)), ((), ())),
                            preferred_element_type=jnp.float32)
        g = h[:, :F]
        u = h[:, F:]
        act = (g * jax.nn.sigmoid(g)) * u
        o = lax.dot_general(act, w2_ref[0],
                            (((1,), (1,)), ((), ())),
                            preferred_element_type=jnp.float32)
        out_ref[...] = o * ws_ref[...]

    @pl.when(jnp.logical_not(active))
    def _():
        out_ref[...] = jnp.zeros_like(out_ref)


def _expert_gemms(counts, gathered, w13, w2, w_slot):
    grid_spec = pltpu.PrefetchScalarGridSpec(
        num_scalar_prefetch=1,
        grid=(E, NCB),
        in_specs=[
            pl.BlockSpec((BC, D), lambda e, cb, cnt: (
                jnp.where(cb * BC < cnt[e], e * NCB + cb, 0), 0)),
            pl.BlockSpec((1, 2 * F, D), lambda e, cb, cnt: (e, 0, 0)),
            pl.BlockSpec((1, D, F), lambda e, cb, cnt: (e, 0, 0)),
            pl.BlockSpec((BC, 1), lambda e, cb, cnt: (e * NCB + cb, 0)),
        ],
        out_specs=pl.BlockSpec((BC, D), lambda e, cb, cnt: (
            jnp.where(cb * BC < cnt[e], e * NCB + cb, E * NCB), 0)),
    )
    return pl.pallas_call(
        _gemm_body,
        grid_spec=grid_spec,
        out_shape=jax.ShapeDtypeStruct((E * C + BC, D), jnp.float32),
        compiler_params=pltpu.CompilerParams(
            dimension_semantics=("arbitrary", "arbitrary"),
            vmem_limit_bytes=100 * 1024 * 1024,
        ),
    )(counts, gathered, w13, w2, w_slot)


_CTOK = T // NW
_CCH = 8


def _combine_body(oute_hbm, sp_hbm, out_hbm, sp_v, rows0, rows1, out_v,
                  sem0, sem1):
    wid = lax.axis_index("s") * NC + lax.axis_index("c")
    tbase = wid * _CTOK
    pltpu.sync_copy(sp_hbm.at[pl.ds(tbase * K, _CTOK * K)], sp_v)
    bufs = (rows0, rows1)
    sems = (sem0, sem1)
    nch = _CTOK // _CCH
    pltpu.async_copy(
        oute_hbm.at[sp_v.at[pl.ds(0, _CCH * K)]], rows0, sem0)

    def chunk(m, carry):
        for par in range(2):
            ch = 2 * m + par

            @pl.when(ch + 1 < nch)
            def _():
                pltpu.async_copy(
                    oute_hbm.at[sp_v.at[pl.ds((ch + 1) * _CCH * K,
                                              _CCH * K)]],
                    bufs[1 - par], sems[1 - par])

            pltpu.make_async_copy(
                oute_hbm.at[sp_v.at[pl.ds(ch * _CCH * K, _CCH * K)]],
                bufs[par], sems[par]).wait()
            rows_v = bufs[par]

            def lane_body(lb, carry2):
                col = lb * L
                for i in range(_CCH):
                    out_v[i, pl.ds(col, L)] = (
                        rows_v[2 * i, pl.ds(col, L)]
                        + rows_v[2 * i + 1, pl.ds(col, L)])
                return carry2

            lax.fori_loop(0, D // L, lane_body, 0)
            pltpu.sync_copy(out_v,
                            out_hbm.at[pl.ds(tbase + ch * _CCH, _CCH)])
        return carry

    lax.fori_loop(0, _CTOK // _CCH // 2, chunk, 0)


def _combine(out_e, slot_pair):
    mesh = plsc.VectorSubcoreMesh(core_axis_name="c", subcore_axis_name="s")
    f = pl.kernel(
        _combine_body,
        mesh=mesh,
        out_type=jax.ShapeDtypeStruct((T, D), jnp.float32),
        scratch_types=[
            pltpu.VMEM((_CTOK * K,), jnp.int32),
            pltpu.VMEM((_CCH * K, D), jnp.float32),
            pltpu.VMEM((_CCH * K, D), jnp.float32),
            pltpu.VMEM((_CCH, D), jnp.float32),
            pltpu.SemaphoreType.DMA,
            pltpu.SemaphoreType.DMA,
        ],
    )
    return f(out_e, slot_pair)


@jax.jit
def kernel(x, router_w, w13, w2):
    topi, topw = _routing(x, router_w)
    idx_buf, w_slot, slot_pair, counts = _dispatch(topi, topw)
    gathered = _gather(x, idx_buf[:E * C], counts)
    out_e = _expert_gemms(counts, gathered, w13, w2,
                          w_slot[:E * C].reshape(E * C, 1))
    return _combine(out_e, slot_pair)
